# baseline (speedup 1.0000x reference)
import functools

import jax
import jax.numpy as jnp
from jax import lax
from jax.experimental import pallas as pl
from jax.experimental.pallas import tpu as pltpu
from jax.experimental.pallas import tpu_sc as plsc

B = 16384
V = 100000
NC, NS = 2, 16
NW = NC * NS
RNG = 3200
HW = (1664, 1536)
TAIL0 = 99872
TAIL1 = 99968
MM = 384
DUMP = B

_iota16 = None


def _scan(idx_v, mcol, mpos, lo, hi, base, iota):
    def pre(g, c):
        sl = pl.ds(g * 16, 16)
        mpos[sl] = iota + DUMP
        mcol[sl] = jnp.zeros((16,), jnp.int32)
        return c

    lax.fori_loop(0, MM // 16 + 1, pre, 0)

    def body(g, off):
        x = idx_v[g >> 3, pl.ds((g & 7) * 16, 16)]
        m = (x >= lo) & (x < hi)
        plsc.store_compressed(mcol.at[pl.ds(off, 16)], x - base, mask=m)
        plsc.store_compressed(mpos.at[pl.ds(off, 16)],
                              iota + g * 16, mask=m)
        return off + plsc.all_reduce_population_count(m)[0]

    return lax.fori_loop(0, (B // 128) * 8, body, 0)


def _extract(strip, mcol, nm, staging, col0, nrows, iota):
    def body(q, c):
        mc = mcol[pl.ds(q * 16, 16)]
        drow = iota + q * 16
        for f in range(nrows):
            val = plsc.load_gather(strip, [jnp.full((16,), f, jnp.int32), mc])
            plsc.store_scatter(staging,
                               [drow, jnp.full((16,), col0 + f, jnp.int32)],
                               val)
        return c

    lax.fori_loop(0, (nm + 15) >> 4, body, 0)


def _make_call(table_specs, n_out):
    mesh = plsc.VectorSubcoreMesh(core_axis_name="c", subcore_axis_name="s")
    out_type = tuple(jax.ShapeDtypeStruct((B + 128, 128), jnp.float32)
                     for _ in range(n_out))
    scratch = [
        pltpu.VMEM((128, 128), jnp.int32),
        pltpu.VMEM((8, 1664), jnp.float32),
        pltpu.VMEM((MM, 128), jnp.float32),
        pltpu.VMEM((MM + 16,), jnp.int32),
        pltpu.VMEM((MM + 16,), jnp.int32),
        pltpu.VMEM((MM // 128, 128), jnp.int32),
        pltpu.SemaphoreType.DMA,
        pltpu.SemaphoreType.DMA,
    ]

    @functools.partial(
        pl.kernel, mesh=mesh, out_type=out_type, scratch_types=scratch,
        compiler_params=pltpu.CompilerParams(needs_layout_passes=False))
    def body(*refs):
        ranks_hbm = refs[0]
        tbls = refs[1:1 + 2 * len(table_specs)]
        outs = refs[1 + 2 * len(table_specs):1 + 2 * len(table_specs) + n_out]
        (idx_v, strip, staging, mcol, mpos, mpos2, sdma, sout) = \
            refs[1 + 2 * len(table_specs) + n_out:]

        iota = lax.iota(jnp.int32, 16)
        wid = lax.axis_index("s") * NC + lax.axis_index("c")
        own_lo = wid * RNG
        own_hi = jnp.minimum(own_lo + RNG, V)
        c0 = jnp.minimum(own_lo, 96768)

        pltpu.sync_copy(ranks_hbm, idx_v)

        def do_range(lo, hi, base, width, tail):
            nm = _scan(idx_v, mcol, mpos, lo, hi, base, iota)
            for oi in range(n_out):
                for ti, (F, oi_t, col0) in enumerate(table_specs):
                    if oi_t != oi:
                        continue
                    main = tbls[2 * ti]
                    tailr = tbls[2 * ti + 1]
                    for s in range(F // 8):
                        if tail:
                            pltpu.async_copy(
                                tailr.at[pl.ds(s * 8, 8)],
                                strip.at[:, pl.ds(0, 128)], sdma).wait()
                        else:
                            pltpu.async_copy(
                                main.at[pl.ds(s * 8, 8),
                                        pl.ds(base, width)],
                                strip.at[:, pl.ds(0, width)], sdma).wait()
                        _extract(strip, mcol, nm, staging, col0 + s * 8, 8,
                                 iota)
                def cp2(g, c):
                    x = mpos[pl.ds(g * 16, 16)]
                    r = g >> 3
                    mpos2[r, pl.ds((g & 7) * 16, 16)] = x
                    return c
                lax.fori_loop(0, MM // 16, cp2, 0)
                for ch in range(MM // 128):
                    pltpu.async_copy(
                        staging.at[pl.ds(ch * 128, 128)],
                        outs[oi].at[mpos2.at[ch]], sout).wait()

        do_range(jnp.maximum(own_lo, c0), jnp.minimum(own_hi, c0 + HW[0]),
                 c0, HW[0], tail=False)
        do_range(jnp.maximum(own_lo, c0 + HW[0]),
                 jnp.minimum(own_hi, jnp.minimum(c0 + HW[0] + HW[1], TAIL1)),
                 c0 + HW[0], HW[1], tail=False)
        lo3 = jnp.where(own_hi >= V, TAIL1, V + 1)
        do_range(lo3, V, TAIL0, 128, tail=True)

    return body


_CALL_A = None
_CALL_B = None


def _get_calls():
    global _CALL_A, _CALL_B
    if _CALL_A is None:
        _CALL_A = _make_call(((64, 0, 0), (64, 0, 64)), 1)
        _CALL_B = _make_call(((128, 0, 0), (32, 1, 0)), 2)
    return _CALL_A, _CALL_B


@jax.jit
def _run(r2, tb1, tail_b1, tw2, tail_w2, w1f, tail_w1, small, tail_small):
    call_a, call_b = _get_calls()
    o2, = call_a(r2, tb1, tail_b1, tw2, tail_w2)
    o1, o3 = call_b(r2, w1f, tail_w1, small, tail_small)
    return o1, o2, o3


def kernel(ranks, func_w1, func_b1, func_w2, func_b2, meas_loc, meas_w):
    r2 = ranks.astype(jnp.int32).reshape(128, 128)
    tb1 = func_b1.T
    tw2 = jnp.transpose(func_w2, (1, 2, 0)).reshape(64, V)
    twm = meas_w.T
    w1f = jnp.transpose(func_w1, (1, 2, 0)).reshape(128, V)
    locf = jnp.transpose(meas_loc, (1, 2, 0)).reshape(20, V)
    small = jnp.concatenate(
        [locf, twm, func_b2.reshape(1, V),
         jnp.zeros((1, V), jnp.float32)], axis=0)
    tail_b1 = func_b1[TAIL0:].T
    tail_w2 = func_w2[TAIL0:].reshape(128, 64).T
    tail_w1 = jnp.transpose(func_w1[TAIL0:], (1, 2, 0)).reshape(128, 128)
    tail_small = small[:, TAIL0:]

    o1, o2, o3 = _run(r2, tb1, tail_b1, tw2, tail_w2, w1f, tail_w1,
                      small, tail_small)
    return (o1[:B].reshape(B, 64, 2),
            o2[:B, :64],
            o2[:B, 64:].reshape(B, 1, 64),
            o3[:B, 30:31],
            o3[:B, :20].reshape(B, 10, 2),
            o3[:B, 20:30])


# speedup vs baseline: 1.4009x; 1.4009x over previous
import functools

import jax
import jax.numpy as jnp
from jax import lax
from jax.experimental import pallas as pl
from jax.experimental.pallas import tpu as pltpu
from jax.experimental.pallas import tpu_sc as plsc

B = 16384
V = 100000
NC, NS = 2, 16
NW = NC * NS
RNG = 3200
HW = (1664, 1536)
TAIL0 = 99872
TAIL1 = 99968
MM = 384
DUMP = B


def _scan(idx_v, mcol, mpos, lo, hi, base, iota):
    def pre(g, c):
        sl = pl.ds(g * 16, 16)
        mpos[sl] = iota + DUMP
        mcol[sl] = jnp.zeros((16,), jnp.int32)
        return c

    lax.fori_loop(0, MM // 16 + 1, pre, 0)

    def body(g4, off):
        for j in range(4):
            g = g4 * 4 + j
            x = idx_v[g >> 3, pl.ds((g & 7) * 16, 16)]
            m = (x >= lo) & (x < hi)
            plsc.store_compressed(mcol.at[pl.ds(off, 16)], x - base, mask=m)
            plsc.store_compressed(mpos.at[pl.ds(off, 16)],
                                  iota + g * 16, mask=m)
            off = off + plsc.all_reduce_population_count(m)[0]
        return off

    return lax.fori_loop(0, (B // 128) * 8 // 4, body, 0)


def _extract(strip, mcol, nm, staging, col0, iota):
    def body(q, c):
        mc = mcol[pl.ds(q * 16, 16)]
        drow = iota + q * 16
        for f in range(8):
            val = plsc.load_gather(strip, [jnp.full((16,), f, jnp.int32), mc])
            plsc.store_scatter(staging,
                               [drow, jnp.full((16,), col0 + f, jnp.int32)],
                               val)
        return c

    lax.fori_loop(0, (nm + 15) >> 4, body, 0)


def _make_call(table_specs, n_out):
    mesh = plsc.VectorSubcoreMesh(core_axis_name="c", subcore_axis_name="s")
    out_type = tuple(jax.ShapeDtypeStruct((B + 128, 128), jnp.float32)
                     for _ in range(n_out))
    scratch = [
        pltpu.VMEM((128, 128), jnp.int32),
        pltpu.VMEM((8, 1664), jnp.float32),
        pltpu.VMEM((8, 1664), jnp.float32),
        pltpu.VMEM((MM, 128), jnp.float32),
        pltpu.VMEM((MM + 16,), jnp.int32),
        pltpu.VMEM((MM + 16,), jnp.int32),
        pltpu.VMEM((MM // 128, 128), jnp.int32),
        pltpu.SemaphoreType.DMA,
        pltpu.SemaphoreType.DMA,
    ]

    @functools.partial(
        pl.kernel, mesh=mesh, out_type=out_type, scratch_types=scratch,
        compiler_params=pltpu.CompilerParams(needs_layout_passes=False))
    def body(*refs):
        nt = len(table_specs)
        ranks_hbm = refs[0]
        tbls = refs[1:1 + 2 * nt]
        outs = refs[1 + 2 * nt:1 + 2 * nt + n_out]
        (idx_v, strip0, strip1, staging, mcol, mpos, mpos2, sdma, sout) = \
            refs[1 + 2 * nt + n_out:]
        strips = (strip0, strip1)

        iota = lax.iota(jnp.int32, 16)
        wid = lax.axis_index("s") * NC + lax.axis_index("c")
        own_lo = wid * RNG
        own_hi = jnp.minimum(own_lo + RNG, V)
        c0 = jnp.minimum(own_lo, 96768)

        pltpu.sync_copy(ranks_hbm, idx_v)

        def do_range(lo, hi, base, width, tail):
            @pl.when(hi > lo)
            def _():
                nm = _scan(idx_v, mcol, mpos, lo, hi, base, iota)

                def cp2(g, c):
                    mpos2[g >> 3, pl.ds((g & 7) * 16, 16)] = \
                        mpos[pl.ds(g * 16, 16)]
                    return c

                lax.fori_loop(0, MM // 16, cp2, 0)

                items = []
                for oi in range(n_out):
                    for ti, (F, oi_t, col0) in enumerate(table_specs):
                        if oi_t != oi:
                            continue
                        for s in range(F // 8):
                            items.append((ti, col0 + s * 8, oi, s * 8))
                last_of_out = {}
                first_of_out = {}
                for k, it in enumerate(items):
                    last_of_out[it[2]] = k
                    first_of_out.setdefault(it[2], k)

                def fire(k, buf):
                    ti, _, _, ro = items[k]
                    if tail:
                        src = tbls[2 * ti + 1].at[pl.ds(ro, 8)]
                    else:
                        src = tbls[2 * ti].at[pl.ds(ro, 8),
                                              pl.ds(base, width)]
                    return pltpu.async_copy(
                        src, buf.at[:, pl.ds(0, width)], sdma)

                cps = {0: fire(0, strips[0])}
                pending = []
                for k, it in enumerate(items):
                    if k + 1 < len(items):
                        cps[k + 1] = fire(k + 1, strips[(k + 1) % 2])
                    cps[k].wait()
                    if first_of_out[it[2]] == k and pending:
                        for c in pending:
                            c.wait()
                        pending.clear()
                    _extract(strips[k % 2], mcol, nm, staging, it[1], iota)
                    if last_of_out[it[2]] == k:
                        for ch in range(MM // 128):
                            pending.append(pltpu.async_copy(
                                staging.at[pl.ds(ch * 128, 128)],
                                outs[it[2]].at[mpos2.at[ch]], sout))
                for c in pending:
                    c.wait()

        do_range(jnp.maximum(own_lo, c0), jnp.minimum(own_hi, c0 + HW[0]),
                 c0, HW[0], tail=False)
        do_range(jnp.maximum(own_lo, c0 + HW[0]),
                 jnp.minimum(own_hi, jnp.minimum(c0 + HW[0] + HW[1], TAIL1)),
                 c0 + HW[0], HW[1], tail=False)
        do_range(jnp.where(own_hi >= V, TAIL1, V + 1), own_hi, TAIL0, 128,
                 tail=True)

    return body


_CALLS = None


def _get_calls():
    global _CALLS
    if _CALLS is None:
        _CALLS = (_make_call(((64, 0, 0), (64, 0, 64)), 1),
                  _make_call(((128, 0, 0), (32, 1, 0)), 2))
    return _CALLS


@jax.jit
def _run(r2, tb1, tail_b1, tw2, tail_w2, w1f, tail_w1, small, tail_small):
    call_a, call_b = _get_calls()
    o2, = call_a(r2, tb1, tail_b1, tw2, tail_w2)
    o1, o3 = call_b(r2, w1f, tail_w1, small, tail_small)
    return o1, o2, o3


def kernel(ranks, func_w1, func_b1, func_w2, func_b2, meas_loc, meas_w):
    r2 = ranks.astype(jnp.int32).reshape(128, 128)
    tb1 = func_b1.T
    tw2 = jnp.transpose(func_w2, (1, 2, 0)).reshape(64, V)
    w1f = jnp.transpose(func_w1, (1, 2, 0)).reshape(128, V)
    locf = jnp.transpose(meas_loc, (1, 2, 0)).reshape(20, V)
    small = jnp.concatenate(
        [locf, meas_w.T, func_b2.reshape(1, V),
         jnp.zeros((1, V), jnp.float32)], axis=0)
    tail_b1 = func_b1[TAIL0:].T
    tail_w2 = func_w2[TAIL0:].reshape(128, 64).T
    tail_w1 = jnp.transpose(func_w1[TAIL0:], (1, 2, 0)).reshape(128, 128)
    tail_small = small[:, TAIL0:]

    o1, o2, o3 = _run(r2, tb1, tail_b1, tw2, tail_w2, w1f, tail_w1,
                      small, tail_small)
    return (o1[:B].reshape(B, 64, 2),
            o2[:B, :64],
            o2[:B, 64:].reshape(B, 1, 64),
            o3[:B, 30:31],
            o3[:B, :20].reshape(B, 10, 2),
            o3[:B, 20:30])
